# baseline (speedup 1.0000x reference)
import functools

import jax
import jax.numpy as jnp
from jax import lax
from jax.experimental import pallas as pl
from jax.experimental.pallas import tpu as pltpu
from jax.experimental.pallas import tpu_sc as plsc

N0, N1 = 64, 64
D0, D1 = 1024, 1024
NC, NS = 2, 16
NW = NC * NS
B = 256
BPW = B // NW
REP = 32
NBUF = 3

_mesh = plsc.VectorSubcoreMesh(core_axis_name="c", subcore_axis_name="s")


@functools.partial(
    pl.kernel,
    mesh=_mesh,
    out_type=jax.ShapeDtypeStruct((B * N0, D0 + D1), jnp.float32),
    scratch_types=[
        pltpu.VMEM((BPW, REP), jnp.int32),
        *[pltpu.VMEM((REP, D1), jnp.float32) for _ in range(NBUF)],
        *[pltpu.SemaphoreType.DMA for _ in range(NBUF)],
        *[pltpu.SemaphoreType.DMA for _ in range(NBUF)],
        pltpu.SemaphoreType.DMA,
    ],
)
def _sc_kernel(idx_hbm, w0_hbm, w1_hbm, out_hbm, idx_v, *scr):
    bufs = scr[:NBUF]
    gsem = scr[NBUF:2 * NBUF]
    osem = scr[2 * NBUF:3 * NBUF]
    dsem = scr[3 * NBUF]
    wid = lax.axis_index("s") * NC + lax.axis_index("c")
    base = wid * BPW
    pltpu.sync_copy(idx_hbm.at[pl.ds(base, BPW)], idx_v)

    gc = [None] * BPW
    wc = [None] * BPW
    dc = [None] * BPW
    for b in range(NBUF):
        gc[b] = pltpu.async_copy(w1_hbm.at[idx_v.at[b]], bufs[b], gsem[b])
    for b in range(BPW):
        x = b % NBUF
        row0 = (base + b) * N0
        dc[b] = pltpu.async_copy(
            w0_hbm, out_hbm.at[pl.ds(row0, N0), pl.ds(0, D0)], dsem
        )
        gc[b].wait()
        wc[b] = (
            pltpu.async_copy(
                bufs[x], out_hbm.at[pl.ds(row0, REP), pl.ds(D0, D1)], osem[x]
            ),
            pltpu.async_copy(
                bufs[x],
                out_hbm.at[pl.ds(row0 + REP, REP), pl.ds(D0, D1)],
                osem[x],
            ),
        )
        if b + NBUF < BPW:
            wc[b][0].wait()
            wc[b][1].wait()
            gc[b + NBUF] = pltpu.async_copy(
                w1_hbm.at[idx_v.at[b + NBUF]], bufs[x], gsem[x]
            )
    for b in range(BPW - NBUF, BPW):
        wc[b][0].wait()
        wc[b][1].wait()
    for b in range(BPW):
        dc[b].wait()


def kernel(position_ids, w0, w1):
    pid = position_ids.reshape(-1).astype(jnp.int32)
    idx_rep = jnp.broadcast_to(pid[:, None], (B, REP))
    out = _sc_kernel(idx_rep, w0.reshape(N0, D0), w1.reshape(N1, D1))
    return out.reshape(*position_ids.shape, N0, D0 + D1)


# speedup vs baseline: 19.9138x; 19.9138x over previous
import functools

import jax
import jax.numpy as jnp
from jax import lax
from jax.experimental import pallas as pl
from jax.experimental.pallas import tpu as pltpu
from jax.experimental.pallas import tpu_sc as plsc

N0, N1 = 64, 64
D0, D1 = 1024, 1024
NC, NS = 2, 16
NW = NC * NS
B = 256
BPW = B // NW
REP = 16
NBUF = 3
WPB = N0 // REP

_mesh = plsc.VectorSubcoreMesh(core_axis_name="c", subcore_axis_name="s")


@functools.partial(
    pl.kernel,
    mesh=_mesh,
    out_type=jax.ShapeDtypeStruct((B * N0, D0 + D1), jnp.float32),
    scratch_types=[
        pltpu.VMEM((BPW, REP), jnp.int32),
        pltpu.VMEM((N0, D0), jnp.float32),
        *[pltpu.VMEM((REP, D1), jnp.float32) for _ in range(NBUF)],
        *[pltpu.SemaphoreType.DMA for _ in range(NBUF)],
        *[pltpu.SemaphoreType.DMA for _ in range(NBUF)],
        pltpu.SemaphoreType.DMA,
    ],
)
def _sc_kernel(idx_hbm, w0_hbm, w1_hbm, out_hbm, idx_v, w0_v, *scr):
    bufs = scr[:NBUF]
    gsem = scr[NBUF:2 * NBUF]
    osem = scr[2 * NBUF:3 * NBUF]
    dsem = scr[3 * NBUF]
    wid = lax.axis_index("s") * NC + lax.axis_index("c")
    base = wid * BPW
    pltpu.sync_copy(idx_hbm.at[pl.ds(base, BPW)], idx_v)
    pltpu.sync_copy(w0_hbm, w0_v)

    gc = [None] * BPW
    wc = [None] * BPW
    dc = [None] * BPW
    for b in range(NBUF):
        gc[b] = pltpu.async_copy(w1_hbm.at[idx_v.at[b]], bufs[b], gsem[b])
    for b in range(BPW):
        x = b % NBUF
        row0 = (base + b) * N0
        dc[b] = pltpu.async_copy(
            w0_v, out_hbm.at[pl.ds(row0, N0), pl.ds(0, D0)], dsem
        )
        gc[b].wait()
        wc[b] = tuple(
            pltpu.async_copy(
                bufs[x],
                out_hbm.at[pl.ds(row0 + r * REP, REP), pl.ds(D0, D1)],
                osem[x],
            )
            for r in range(WPB)
        )
        if b + NBUF < BPW:
            for c in wc[b]:
                c.wait()
            gc[b + NBUF] = pltpu.async_copy(
                w1_hbm.at[idx_v.at[b + NBUF]], bufs[x], gsem[x]
            )
    for b in range(BPW - NBUF, BPW):
        for c in wc[b]:
            c.wait()
    for b in range(BPW):
        dc[b].wait()


def kernel(position_ids, w0, w1):
    pid = position_ids.reshape(-1).astype(jnp.int32)
    idx_rep = jnp.broadcast_to(pid[:, None], (B, REP))
    out = _sc_kernel(idx_rep, w0.reshape(N0, D0), w1.reshape(N1, D1))
    return out.reshape(*position_ids.shape, N0, D0 + D1)
